# baseline (speedup 1.0000x reference)
import functools

import jax
import jax.numpy as jnp
from jax import lax
from jax.experimental import pallas as pl
from jax.experimental.pallas import tpu as pltpu
from jax.experimental.pallas import tpu_sc as plsc

D = 64
SEQ = 50
NC = 2
NS = 16
NW = NC * NS
CHUNK = 128

TWO_PI = 6.2831853071795864769
INV_TWO_PI = 1.0 / TWO_PI

S0 = 9.9945015e-01
S1 = -1.6583844e-01
S2 = 7.9985755e-03
S3 = -1.4774044e-04
C0 = 9.9997109e-01
C1 = -4.9983761e-01
C2 = 4.1522305e-02
C3 = -1.3441069e-03
C4 = 1.9065215e-05


def _sincos(ph):
    t = ph * INV_TWO_PI
    k = (t + 0.5 * lax.sign(t)).astype(jnp.int32).astype(jnp.float32)
    r = ph - k * TWO_PI
    u = r * r
    s = r * (S0 + u * (S1 + u * (S2 + u * S3)))
    c = C0 + u * (C1 + u * (C2 + u * (C3 + u * C4)))
    return s, c


def _body(rows_per_w, n_chunks,
          x_hbm, label_hbm, freq_hbm, phase_hbm,
          real_hbm, imag_hbm,
          idx_v, amp_v, frq_v, bia_v, re_v, im_v, gsem, osem):
    wid = lax.axis_index("s") * NC + lax.axis_index("c")
    base_w = wid * rows_per_w

    def chunk_base(ci):
        return pl.multiple_of(base_w + ci * CHUNK, CHUNK)

    def start_gathers(ci, slot):
        base = chunk_base(ci)
        pltpu.sync_copy(x_hbm.at[pl.ds(base, CHUNK)], idx_v.at[slot])
        pltpu.async_copy(label_hbm.at[idx_v.at[slot]], amp_v.at[slot], gsem.at[slot])
        pltpu.async_copy(freq_hbm.at[idx_v.at[slot]], frq_v.at[slot], gsem.at[slot])
        pltpu.async_copy(phase_hbm.at[idx_v.at[slot]], bia_v.at[slot], gsem.at[slot])

    def wait_gathers(slot):
        pltpu.make_async_copy(label_hbm.at[idx_v.at[slot]], amp_v.at[slot], gsem.at[slot]).wait()
        pltpu.make_async_copy(freq_hbm.at[idx_v.at[slot]], frq_v.at[slot], gsem.at[slot]).wait()
        pltpu.make_async_copy(phase_hbm.at[idx_v.at[slot]], bia_v.at[slot], gsem.at[slot]).wait()

    def wait_stores(slot):
        base0 = chunk_base(0)
        pltpu.make_async_copy(re_v.at[slot], real_hbm.at[pl.ds(base0, CHUNK)], osem.at[slot]).wait()
        pltpu.make_async_copy(im_v.at[slot], imag_hbm.at[pl.ds(base0, CHUNK)], osem.at[slot]).wait()

    start_gathers(0, 0)

    @pl.loop(0, n_chunks, step=2)
    def chunk_loop(ci0):
        for b in range(2):
            ci = ci0 + b
            slot = b

            @pl.when(ci + 1 < n_chunks)
            def _prefetch():
                start_gathers(ci + 1, 1 - slot)

            wait_gathers(slot)

            @pl.when(ci >= 2)
            def _drain():
                wait_stores(slot)

            base = chunk_base(ci)

            pos0 = (lax.rem(base, SEQ) + 1).astype(jnp.float32)

            @pl.loop(0, CHUNK, init_carry=pos0, unroll=2)
            def row_loop(r, pos):
                for q in range(D // 16):
                    sl = pl.ds(q * 16, 16)
                    f = frq_v[slot, r, sl]
                    bb = bia_v[slot, r, sl]
                    a = amp_v[slot, r, sl]
                    s, c = _sincos(pos * f + bb)
                    re_v[slot, r, sl] = a * c
                    im_v[slot, r, sl] = a * s
                return lax.select(pos >= float(SEQ), 1.0, pos + 1.0)

            pltpu.async_copy(re_v.at[slot], real_hbm.at[pl.ds(base, CHUNK)], osem.at[slot])
            pltpu.async_copy(im_v.at[slot], imag_hbm.at[pl.ds(base, CHUNK)], osem.at[slot])

    wait_stores(0)
    wait_stores(1)


def kernel(x, label_w, freq_w, phase_w):
    bsz, seq = x.shape
    n_rows = bsz * seq
    rows_per_w = n_rows // NW
    n_chunks = rows_per_w // CHUNK
    xf = x.reshape(n_rows)

    out_sds = jax.ShapeDtypeStruct((n_rows, D), jnp.float32)
    run = pl.kernel(
        functools.partial(_body, rows_per_w, n_chunks),
        out_type=[out_sds, out_sds],
        mesh=plsc.VectorSubcoreMesh(core_axis_name="c", subcore_axis_name="s"),
        scratch_types=[
            pltpu.VMEM((2, CHUNK), jnp.int32),
            pltpu.VMEM((2, CHUNK, D), jnp.float32),
            pltpu.VMEM((2, CHUNK, D), jnp.float32),
            pltpu.VMEM((2, CHUNK, D), jnp.float32),
            pltpu.VMEM((2, CHUNK, D), jnp.float32),
            pltpu.VMEM((2, CHUNK, D), jnp.float32),
            pltpu.SemaphoreType.DMA((2,)),
            pltpu.SemaphoreType.DMA((2,)),
        ],
        compiler_params=pltpu.CompilerParams(use_tc_tiling_on_sc=False),
    )
    real, imag = run(xf, label_w, freq_w, phase_w)
    return real.reshape(bsz, seq, D), imag.reshape(bsz, seq, D)


# speedup vs baseline: 1.9461x; 1.9461x over previous
import functools

import jax
import jax.numpy as jnp
from jax import lax
from jax.experimental import pallas as pl
from jax.experimental.pallas import tpu as pltpu
from jax.experimental.pallas import tpu_sc as plsc

D = 64
SEQ = 50
NC = 2
NS = 16
NW = NC * NS
CHUNK = 128

TWO_PI = 6.2831853071795864769
INV_TWO_PI = 1.0 / TWO_PI

S0 = 9.9945015e-01
S1 = -1.6583844e-01
S2 = 7.9985755e-03
S3 = -1.4774044e-04
C0 = 9.9997109e-01
C1 = -4.9983761e-01
C2 = 4.1522305e-02
C3 = -1.3441069e-03
C4 = 1.9065215e-05


def _sincos(ph):
    t = ph * INV_TWO_PI
    k = (t + 0.5 * lax.sign(t)).astype(jnp.int32).astype(jnp.float32)
    r = ph - k * TWO_PI
    u = r * r
    s = r * (S0 + u * (S1 + u * (S2 + u * S3)))
    c = C0 + u * (C1 + u * (C2 + u * (C3 + u * C4)))
    return s, c


def _body(rows_per_w, n_chunks,
          x_hbm, label_hbm, freq_hbm, phase_hbm,
          real_hbm, imag_hbm,
          idx_v, amp_v, frq_v, bia_v, re_v, im_v, gsem, osem):
    wid = lax.axis_index("s") * NC + lax.axis_index("c")
    base_w = wid * rows_per_w

    def chunk_base(ci):
        return pl.multiple_of(base_w + ci * CHUNK, CHUNK)

    def start_gathers(ci, slot):
        base = chunk_base(ci)
        pltpu.sync_copy(x_hbm.at[pl.ds(base, CHUNK)], idx_v.at[slot])
        pltpu.async_copy(label_hbm.at[idx_v.at[slot]], amp_v.at[slot], gsem.at[slot])
        pltpu.async_copy(freq_hbm.at[idx_v.at[slot]], frq_v.at[slot], gsem.at[slot])
        pltpu.async_copy(phase_hbm.at[idx_v.at[slot]], bia_v.at[slot], gsem.at[slot])

    def wait_gathers(slot):
        pltpu.make_async_copy(label_hbm.at[idx_v.at[slot]], amp_v.at[slot], gsem.at[slot]).wait()
        pltpu.make_async_copy(freq_hbm.at[idx_v.at[slot]], frq_v.at[slot], gsem.at[slot]).wait()
        pltpu.make_async_copy(phase_hbm.at[idx_v.at[slot]], bia_v.at[slot], gsem.at[slot]).wait()

    def wait_stores(slot):
        base0 = chunk_base(0)
        pltpu.make_async_copy(re_v.at[slot], real_hbm.at[pl.ds(base0, CHUNK)], osem.at[slot]).wait()
        pltpu.make_async_copy(im_v.at[slot], imag_hbm.at[pl.ds(base0, CHUNK)], osem.at[slot]).wait()

    start_gathers(0, 0)

    @pl.loop(0, n_chunks, step=2)
    def chunk_loop(ci0):
        for b in range(2):
            ci = ci0 + b
            slot = b

            @pl.when(ci + 1 < n_chunks)
            def _prefetch():
                start_gathers(ci + 1, 1 - slot)

            wait_gathers(slot)

            @pl.when(ci >= 2)
            def _drain():
                wait_stores(slot)

            base = chunk_base(ci)

            pos0 = (lax.rem(base, SEQ) + 1).astype(jnp.float32)

            @pl.loop(0, CHUNK, init_carry=pos0)
            def row_loop(r, pos):
                for q in range(D // 16):
                    sl = pl.ds(q * 16, 16)
                    f = frq_v[slot, r, sl]
                    bb = bia_v[slot, r, sl]
                    a = amp_v[slot, r, sl]
                    s, c = _sincos(pos * f + bb)
                    re_v[slot, r, sl] = a * c
                    im_v[slot, r, sl] = a * s
                return lax.select(pos >= float(SEQ), 1.0, pos + 1.0)

            pltpu.async_copy(re_v.at[slot], real_hbm.at[pl.ds(base, CHUNK)], osem.at[slot])
            pltpu.async_copy(im_v.at[slot], imag_hbm.at[pl.ds(base, CHUNK)], osem.at[slot])

    wait_stores(0)
    wait_stores(1)


def kernel(x, label_w, freq_w, phase_w):
    bsz, seq = x.shape
    n_rows = bsz * seq
    rows_per_w = n_rows // NW
    n_chunks = rows_per_w // CHUNK
    xf = x.reshape(n_rows)

    out_sds = jax.ShapeDtypeStruct((n_rows, D), jnp.float32)
    run = pl.kernel(
        functools.partial(_body, rows_per_w, n_chunks),
        out_type=[out_sds, out_sds],
        mesh=plsc.VectorSubcoreMesh(core_axis_name="c", subcore_axis_name="s"),
        scratch_types=[
            pltpu.VMEM((2, CHUNK), jnp.int32),
            pltpu.VMEM((2, CHUNK, D), jnp.float32),
            pltpu.VMEM((2, CHUNK, D), jnp.float32),
            pltpu.VMEM((2, CHUNK, D), jnp.float32),
            pltpu.VMEM((2, CHUNK, D), jnp.float32),
            pltpu.VMEM((2, CHUNK, D), jnp.float32),
            pltpu.SemaphoreType.DMA((2,)),
            pltpu.SemaphoreType.DMA((2,)),
        ],
        compiler_params=pltpu.CompilerParams(use_tc_tiling_on_sc=False),
    )
    real, imag = run(xf, label_w, freq_w, phase_w)
    return real.reshape(bsz, seq, D), imag.reshape(bsz, seq, D)


# speedup vs baseline: 1.9508x; 1.0024x over previous
import functools

import jax
import jax.numpy as jnp
from jax import lax
from jax.experimental import pallas as pl
from jax.experimental.pallas import tpu as pltpu
from jax.experimental.pallas import tpu_sc as plsc

D = 64
SEQ = 50
NC = 2
NS = 16
NW = NC * NS
CHUNK = 128

TWO_PI = 6.2831853071795864769
INV_TWO_PI = 1.0 / TWO_PI

S0 = 9.9945015e-01
S1 = -1.6583844e-01
S2 = 7.9985755e-03
S3 = -1.4774044e-04
C0 = 9.9997109e-01
C1 = -4.9983761e-01
C2 = 4.1522305e-02
C3 = -1.3441069e-03
C4 = 1.9065215e-05


def _sincos(ph):
    t = ph * INV_TWO_PI
    k = (t + 0.5 * lax.sign(t)).astype(jnp.int32).astype(jnp.float32)
    r = ph - k * TWO_PI
    u = r * r
    s = r * (S0 + u * (S1 + u * (S2 + u * S3)))
    c = C0 + u * (C1 + u * (C2 + u * (C3 + u * C4)))
    return s, c


def _body(rows_per_w, n_chunks,
          x_hbm, label_hbm, freq_hbm, phase_hbm,
          real_hbm, imag_hbm,
          idx_v, amp_v, frq_v, bia_v, re_v, im_v, gsem, osem):
    wid = lax.axis_index("s") * NC + lax.axis_index("c")
    base_w = wid * rows_per_w

    def chunk_base(ci):
        return pl.multiple_of(base_w + ci * CHUNK, CHUNK)

    def start_gathers(ci, slot):
        base = chunk_base(ci)
        pltpu.sync_copy(x_hbm.at[pl.ds(base, CHUNK)], idx_v.at[slot])
        pltpu.async_copy(label_hbm.at[idx_v.at[slot]], amp_v.at[slot], gsem.at[slot])
        pltpu.async_copy(freq_hbm.at[idx_v.at[slot]], frq_v.at[slot], gsem.at[slot])
        pltpu.async_copy(phase_hbm.at[idx_v.at[slot]], bia_v.at[slot], gsem.at[slot])

    def wait_gathers(slot):
        pltpu.make_async_copy(label_hbm.at[idx_v.at[slot]], amp_v.at[slot], gsem.at[slot]).wait()
        pltpu.make_async_copy(freq_hbm.at[idx_v.at[slot]], frq_v.at[slot], gsem.at[slot]).wait()
        pltpu.make_async_copy(phase_hbm.at[idx_v.at[slot]], bia_v.at[slot], gsem.at[slot]).wait()

    def wait_stores(slot):
        base0 = chunk_base(0) * D
        pltpu.make_async_copy(re_v.at[slot], real_hbm.at[pl.ds(base0, CHUNK * D)], osem.at[slot]).wait()
        pltpu.make_async_copy(im_v.at[slot], imag_hbm.at[pl.ds(base0, CHUNK * D)], osem.at[slot]).wait()

    start_gathers(0, 0)

    @pl.loop(0, n_chunks, step=2)
    def chunk_loop(ci0):
        for b in range(2):
            ci = ci0 + b
            slot = b

            @pl.when(ci + 1 < n_chunks)
            def _prefetch():
                start_gathers(ci + 1, 1 - slot)

            wait_gathers(slot)

            @pl.when(ci >= 2)
            def _drain():
                wait_stores(slot)

            base = chunk_base(ci)

            pos0 = (lax.rem(base, SEQ) + 1).astype(jnp.float32)

            @pl.loop(0, CHUNK, init_carry=pos0)
            def row_loop(r, pos):
                for q in range(D // 16):
                    sl = pl.ds(q * 16, 16)
                    slo = pl.ds(r * D + q * 16, 16)
                    f = frq_v[slot, r, sl]
                    bb = bia_v[slot, r, sl]
                    a = amp_v[slot, r, sl]
                    s, c = _sincos(pos * f + bb)
                    re_v[slot, slo] = a * c
                    im_v[slot, slo] = a * s
                return lax.select(pos >= float(SEQ), 1.0, pos + 1.0)

            pltpu.async_copy(re_v.at[slot], real_hbm.at[pl.ds(base * D, CHUNK * D)], osem.at[slot])
            pltpu.async_copy(im_v.at[slot], imag_hbm.at[pl.ds(base * D, CHUNK * D)], osem.at[slot])

    wait_stores(0)
    wait_stores(1)


def kernel(x, label_w, freq_w, phase_w):
    bsz, seq = x.shape
    n_rows = bsz * seq
    rows_per_w = n_rows // NW
    n_chunks = rows_per_w // CHUNK
    xf = x.reshape(n_rows)

    out_sds = jax.ShapeDtypeStruct((n_rows * D,), jnp.float32)
    run = pl.kernel(
        functools.partial(_body, rows_per_w, n_chunks),
        out_type=[out_sds, out_sds],
        mesh=plsc.VectorSubcoreMesh(core_axis_name="c", subcore_axis_name="s"),
        scratch_types=[
            pltpu.VMEM((2, CHUNK), jnp.int32),
            pltpu.VMEM((2, CHUNK, D), jnp.float32),
            pltpu.VMEM((2, CHUNK, D), jnp.float32),
            pltpu.VMEM((2, CHUNK, D), jnp.float32),
            pltpu.VMEM((2, CHUNK * D), jnp.float32),
            pltpu.VMEM((2, CHUNK * D), jnp.float32),
            pltpu.SemaphoreType.DMA((2,)),
            pltpu.SemaphoreType.DMA((2,)),
        ],
        compiler_params=pltpu.CompilerParams(use_tc_tiling_on_sc=False),
    )
    real, imag = run(xf, label_w, freq_w, phase_w)
    return real.reshape(bsz, seq, D), imag.reshape(bsz, seq, D)
